# baseline (speedup 1.0000x reference)
import jax
import jax.numpy as jnp
from jax.experimental import pallas as pl

_ROW_BLOCK = 2048


def _stage_kernel(res_ref, x2_ref, cb_ref, c2_ref,
                  q_ref, resout_ref, nn_ref, counts_ref, sse_ref):
    B = res_ref.shape[0]
    K = cb_ref.shape[0]

    @pl.when(pl.program_id(0) == 0)
    def _init():
        counts_ref[...] = jnp.zeros_like(counts_ref)
        sse_ref[...] = jnp.zeros_like(sse_ref)

    residual = res_ref[...]
    cb = cb_ref[...]
    x2 = x2_ref[...]
    c2 = c2_ref[...]
    dots = jax.lax.dot_general(
        residual.astype(jnp.bfloat16), cb.astype(jnp.bfloat16),
        (((1,), (1,)), ((), ())),
        preferred_element_type=jnp.float32)
    dists = x2 - 2.0 * dots + c2
    m = jnp.min(dists, axis=1, keepdims=True)
    iota = jax.lax.broadcasted_iota(jnp.int32, (B, K), 1)
    nn = jnp.min(jnp.where(dists == m, iota, K), axis=1)
    nn_ref[...] = nn[:, None]
    onehot = (iota == nn[:, None]).astype(jnp.float32)
    counts_ref[...] += jnp.sum(onehot, axis=0).astype(jnp.int32)[None, :]
    e = jnp.dot(onehot, cb, preferred_element_type=jnp.float32,
                precision=jax.lax.Precision.HIGHEST)
    diff = residual - e
    sse_ref[...] += jnp.sum(diff * diff)
    q = residual + (e - residual)
    q_ref[...] = q
    resout_ref[...] = residual - q


def _stage(residual, x2, cb, c2):
    n, d = residual.shape
    k = cb.shape[0]
    return pl.pallas_call(
        _stage_kernel,
        grid=(n // _ROW_BLOCK,),
        in_specs=[
            pl.BlockSpec((_ROW_BLOCK, d), lambda r: (r, 0)),
            pl.BlockSpec((_ROW_BLOCK, 1), lambda r: (r, 0)),
            pl.BlockSpec((k, d), lambda r: (0, 0)),
            pl.BlockSpec((1, k), lambda r: (0, 0)),
        ],
        out_specs=[
            pl.BlockSpec((_ROW_BLOCK, d), lambda r: (r, 0)),
            pl.BlockSpec((_ROW_BLOCK, d), lambda r: (r, 0)),
            pl.BlockSpec((_ROW_BLOCK, 1), lambda r: (r, 0)),
            pl.BlockSpec((1, k), lambda r: (0, 0)),
            pl.BlockSpec((1, 1), lambda r: (0, 0)),
        ],
        out_shape=[
            jax.ShapeDtypeStruct((n, d), jnp.float32),
            jax.ShapeDtypeStruct((n, d), jnp.float32),
            jax.ShapeDtypeStruct((n, 1), jnp.int32),
            jax.ShapeDtypeStruct((1, k), jnp.int32),
            jax.ShapeDtypeStruct((1, 1), jnp.float32),
        ],
    )(residual, x2, cb, c2)


def kernel(inputs, codebooks):
    batch, tokens, d = inputs.shape
    num_q, num_centroids, _ = codebooks.shape
    n = batch * tokens
    flat = jnp.reshape(inputs, (n, d))

    residual = flat
    quantized = jnp.zeros_like(flat)
    loss = jnp.float32(0.0)
    denom = jnp.float32(n * d)
    nn_list, counts_list = [], []
    for i in range(num_q):
        cb = codebooks[i]
        c2 = jnp.sum(cb * cb, axis=1)[None, :]
        x2 = jnp.sum(residual * residual, axis=1, keepdims=True)
        q, residual, nn, counts, sse = _stage(residual, x2, cb, c2)
        quantized = quantized + q
        loss = loss + 1.25 * (sse[0, 0] / denom)
        nn_list.append(nn[:, 0])
        counts_list.append(counts[0])

    quantized = jnp.reshape(quantized, inputs.shape)
    qloss_arr = jnp.full(inputs.shape[:-1] + (1,), loss)
    nn_out = jnp.reshape(jnp.stack(nn_list, axis=0), (num_q, batch, tokens))
    cbs = jnp.reshape(codebooks, (num_q * num_centroids, d))
    counts_out = jnp.stack(counts_list, axis=0)
    return (quantized, qloss_arr, nn_out, cbs, counts_out)


# speedup vs baseline: 1.2646x; 1.2646x over previous
import jax
import jax.numpy as jnp
from jax.experimental import pallas as pl

_ROW_BLOCK = 2048


def _stage_kernel(res_ref, x2_ref, cb_ref, cbp_ref, c2_ref,
                  q_ref, resout_ref, nn_ref, counts_ref, sse_ref):
    B = res_ref.shape[0]
    K = cbp_ref.shape[1]

    @pl.when(pl.program_id(0) == 0)
    def _init():
        counts_ref[...] = jnp.zeros_like(counts_ref)
        sse_ref[...] = jnp.zeros_like(sse_ref)

    residual = res_ref[...]
    x2 = x2_ref[...]
    c2 = c2_ref[...]
    dots = jax.lax.dot_general(
        residual.astype(jnp.bfloat16), cb_ref[...].astype(jnp.bfloat16),
        (((1,), (1,)), ((), ())),
        preferred_element_type=jnp.float32)
    dists = x2 - 2.0 * dots + c2
    m = jnp.min(dists, axis=1, keepdims=True)
    iota = jax.lax.broadcasted_iota(jnp.int32, (B, K), 1)
    nn = jnp.min(jnp.where(dists == m, iota, K), axis=1)
    nn_ref[...] = nn[:, None]
    onehot = (iota == nn[:, None]).astype(jnp.float32)
    counts_ref[...] += jnp.sum(onehot, axis=0).astype(jnp.int32)[None, :]
    ohb = onehot.astype(jnp.bfloat16)
    e_hi = jnp.dot(ohb, cbp_ref[0], preferred_element_type=jnp.float32)
    e_mid = jnp.dot(ohb, cbp_ref[1], preferred_element_type=jnp.float32)
    e_lo = jnp.dot(ohb, cbp_ref[2], preferred_element_type=jnp.float32)
    e = (e_hi + e_mid) + e_lo
    diff = residual - e
    sse_ref[...] += jnp.sum(diff * diff)
    q = residual + (e - residual)
    q_ref[...] = q
    resout_ref[...] = residual - q


def _stage(residual, x2, cb, cb_parts, c2):
    n, d = residual.shape
    k = cb_parts.shape[1]
    return pl.pallas_call(
        _stage_kernel,
        grid=(n // _ROW_BLOCK,),
        in_specs=[
            pl.BlockSpec((_ROW_BLOCK, d), lambda r: (r, 0)),
            pl.BlockSpec((_ROW_BLOCK, 1), lambda r: (r, 0)),
            pl.BlockSpec((k, d), lambda r: (0, 0)),
            pl.BlockSpec((3, k, d), lambda r: (0, 0, 0)),
            pl.BlockSpec((1, k), lambda r: (0, 0)),
        ],
        out_specs=[
            pl.BlockSpec((_ROW_BLOCK, d), lambda r: (r, 0)),
            pl.BlockSpec((_ROW_BLOCK, d), lambda r: (r, 0)),
            pl.BlockSpec((_ROW_BLOCK, 1), lambda r: (r, 0)),
            pl.BlockSpec((1, k), lambda r: (0, 0)),
            pl.BlockSpec((1, 1), lambda r: (0, 0)),
        ],
        out_shape=[
            jax.ShapeDtypeStruct((n, d), jnp.float32),
            jax.ShapeDtypeStruct((n, d), jnp.float32),
            jax.ShapeDtypeStruct((n, 1), jnp.int32),
            jax.ShapeDtypeStruct((1, k), jnp.int32),
            jax.ShapeDtypeStruct((1, 1), jnp.float32),
        ],
    )(residual, x2, cb, cb_parts, c2)


def kernel(inputs, codebooks):
    batch, tokens, d = inputs.shape
    num_q, num_centroids, _ = codebooks.shape
    n = batch * tokens
    flat = jnp.reshape(inputs, (n, d))

    mask = jnp.uint32(0xFFFF0000)
    u = jax.lax.bitcast_convert_type(codebooks, jnp.uint32)
    hi_f = jax.lax.bitcast_convert_type(u & mask, jnp.float32)
    r1 = codebooks - hi_f
    u1 = jax.lax.bitcast_convert_type(r1, jnp.uint32)
    mid_f = jax.lax.bitcast_convert_type(u1 & mask, jnp.float32)
    lo_f = r1 - mid_f
    parts = jnp.stack([hi_f.astype(jnp.bfloat16), mid_f.astype(jnp.bfloat16),
                       lo_f.astype(jnp.bfloat16)], axis=1)

    residual = flat
    quantized = jnp.zeros_like(flat)
    loss = jnp.float32(0.0)
    denom = jnp.float32(n * d)
    nn_list, counts_list = [], []
    for i in range(num_q):
        cb = codebooks[i]
        c2 = jnp.sum(cb * cb, axis=1)[None, :]
        x2 = jnp.sum(residual * residual, axis=1, keepdims=True)
        q, residual, nn, counts, sse = _stage(residual, x2, cb, parts[i], c2)
        quantized = quantized + q
        loss = loss + 1.25 * (sse[0, 0] / denom)
        nn_list.append(nn[:, 0])
        counts_list.append(counts[0])

    quantized = jnp.reshape(quantized, inputs.shape)
    qloss_arr = jnp.full(inputs.shape[:-1] + (1,), loss)
    nn_out = jnp.reshape(jnp.stack(nn_list, axis=0), (num_q, batch, tokens))
    cbs = jnp.reshape(codebooks, (num_q * num_centroids, d))
    counts_out = jnp.stack(counts_list, axis=0)
    return (quantized, qloss_arr, nn_out, cbs, counts_out)
